# baseline (device time: 15172 ns/iter reference)
import jax
import jax.numpy as jnp
from jax import lax
from jax.experimental import pallas as pl
from jax.experimental.pallas import tpu as pltpu

N_DEV = 4
B, SQ, SKV, D_MODEL = 2, 128, 128, 512
HQ_LOCAL, DH = 4, 64
D_LOCAL = HQ_LOCAL * DH


def _body(x_ref, wq_ref, k_ref, v_ref, wo_ref, out_ref,
          send_ref, recv_ref, send_sems, recv_sems):
    my_pos = lax.axis_index("i")
    p1 = my_pos ^ 1
    p2 = my_pos ^ 2

    barrier_sem = pltpu.get_barrier_semaphore()
    for p in (p1, p2):
        pl.semaphore_signal(
            barrier_sem, inc=1,
            device_id=(p,), device_id_type=pl.DeviceIdType.MESH,
        )

    qb = lax.broadcasted_iota(jnp.int32, (SQ, SKV), 0) // 64
    kb = lax.broadcasted_iota(jnp.int32, (SQ, SKV), 1) // 64
    mask = qb == kb

    wq = wq_ref[:].astype(jnp.bfloat16)
    wo = wo_ref[:].astype(jnp.bfloat16)

    def partial_for_batch(b):
        xb = x_ref[b].astype(jnp.bfloat16)
        q_all = lax.dot_general(
            xb, wq, (((1,), (0,)), ((), ())),
            preferred_element_type=jnp.float32,
        )
        q_all = (q_all * 0.125).astype(jnp.bfloat16)
        head_cols = []
        for h in range(HQ_LOCAL):
            q = q_all[:, h * DH:(h + 1) * DH]
            k = k_ref[b, h].astype(jnp.bfloat16)
            v = v_ref[b, h].astype(jnp.bfloat16)
            s = lax.dot_general(
                q, k, (((1,), (1,)), ((), ())),
                preferred_element_type=jnp.float32,
            )
            s = jnp.where(mask, s, -1e9)
            m = jnp.max(s, axis=-1, keepdims=True)
            w = jnp.exp(s - m)
            w = w / jnp.sum(w, axis=-1, keepdims=True)
            head_cols.append(lax.dot_general(
                w.astype(jnp.bfloat16), v, (((1,), (0,)), ((), ())),
                preferred_element_type=jnp.float32,
            ))
        ctx = jnp.concatenate(head_cols, axis=1).astype(jnp.bfloat16)
        return lax.dot_general(
            ctx, wo, (((1,), (0,)), ((), ())),
            preferred_element_type=jnp.float32,
        )

    def exchange(slot, peer):
        return pltpu.make_async_remote_copy(
            src_ref=send_ref.at[slot],
            dst_ref=recv_ref.at[slot],
            send_sem=send_sems.at[slot],
            recv_sem=recv_sems.at[slot],
            device_id=(peer,),
            device_id_type=pl.DeviceIdType.MESH,
        )

    partials, r1 = [], []
    for b in range(B):
        pb = partial_for_batch(b)
        partials.append(pb)
        send_ref[b] = pb.astype(jnp.bfloat16)
        if b == 0:
            pl.semaphore_wait(barrier_sem, 2)
        r1.append(exchange(b, p1))
        r1[b].start()

    accs, r2 = [], []
    for b in range(B):
        r1[b].wait_recv()
        acc = partials[b] + recv_ref[b].astype(jnp.float32)
        accs.append(acc)
        send_ref[B + b] = acc.astype(jnp.bfloat16)
        r2.append(exchange(B + b, p2))
        r2[b].start()

    for b in range(B):
        r2[b].wait_recv()
        out_ref[b] = accs[b] + recv_ref[B + b].astype(jnp.float32)

    for b in range(B):
        r1[b].wait_send()
        r2[b].wait_send()


def kernel(x, Wq, K_ext, V_ext, Wo):
    my_pos = lax.axis_index("i")
    k_loc = jnp.transpose(
        lax.dynamic_slice_in_dim(K_ext, my_pos * HQ_LOCAL, HQ_LOCAL, axis=2),
        (0, 2, 1, 3),
    )
    v_loc = jnp.transpose(
        lax.dynamic_slice_in_dim(V_ext, my_pos * HQ_LOCAL, HQ_LOCAL, axis=2),
        (0, 2, 1, 3),
    )
    return pl.pallas_call(
        _body,
        out_shape=jax.ShapeDtypeStruct((B, SQ, D_MODEL), jnp.float32),
        in_specs=[pl.BlockSpec(memory_space=pltpu.VMEM)] * 5,
        out_specs=pl.BlockSpec(memory_space=pltpu.VMEM),
        scratch_shapes=[
            pltpu.VMEM((2 * B, SQ, D_MODEL), jnp.bfloat16),
            pltpu.VMEM((2 * B, SQ, D_MODEL), jnp.bfloat16),
            pltpu.SemaphoreType.DMA((2 * B,)),
            pltpu.SemaphoreType.DMA((2 * B,)),
        ],
        compiler_params=pltpu.CompilerParams(collective_id=0),
    )(x, Wq, k_loc, v_loc, Wo)
